# Initial kernel scaffold; baseline (speedup 1.0000x reference)
import jax
import jax.numpy as jnp
from jax.experimental import pallas as pl


def kernel(x, edge_idx, conv1_w, conv1_b, bn1_g, bn1_b, conv2_w, conv2_b, bn2_g, bn2_b):
    raise NotImplementedError("write your pallas kernel here")


# speedup vs baseline: 3570.8103x; 3570.8103x over previous
import functools

import jax
import jax.numpy as jnp
from jax import lax
from jax.experimental import pallas as pl
from jax.experimental.pallas import tpu as pltpu
from jax.experimental.pallas import tpu_sc as plsc

B, C, N, K = 16, 96, 1024, 16
C2 = 2 * C
COUT = 96
NSAMP = B * N
NW = 32
NODES_PER_W = (B * N) // NW
GN = 128
NSUB = GN // 16
NGROUPS = NODES_PER_W // GN


def _sc_body(xc_hbm, e0_hbm, e1_hbm, msg_hbm, table, idx0, idx1, msgbuf):
    cid = lax.axis_index("c")
    sid = lax.axis_index("s")
    wid = sid * 2 + cid
    b = wid // 2
    half = wid % 2
    lanes = lax.iota(jnp.int32, 16)

    pltpu.sync_copy(xc_hbm.at[b], table)

    def group(g, carry):
        n0 = half * NODES_PER_W + g * GN
        pltpu.sync_copy(e0_hbm.at[b, pl.ds(n0 * K, GN * K)], idx0)
        pltpu.sync_copy(e1_hbm.at[b, pl.ds(n0 * K, GN * K)], idx1)

        def sub(sg, sc_):
            nodes = sg * 16 + lanes
            rs = [plsc.load_gather(idx0, [nodes * K + k]) for k in range(K)]
            rd = [plsc.load_gather(idx1, [nodes * K + k]) for k in range(K)]

            def cbody(c, cc):
                cvec = jnp.full((16,), c, jnp.int32)
                acc = jnp.full((16,), -jnp.inf, jnp.float32)
                for k in range(K):
                    s = plsc.load_gather(table, [cvec, rs[k]])
                    d = plsc.load_gather(table, [cvec, rd[k]])
                    acc = jnp.maximum(acc, s - d)
                plsc.store_scatter(msgbuf, [cvec, nodes], acc)
                return cc

            lax.fori_loop(0, C, cbody, 0)
            return sc_

        lax.fori_loop(0, NSUB, sub, 0)
        pltpu.sync_copy(msgbuf, msg_hbm.at[b, :, pl.ds(n0, GN)])
        return carry

    lax.fori_loop(0, NGROUPS, group, 0)


@functools.cache
def _sc_msg():
    return pl.kernel(
        _sc_body,
        out_type=jax.ShapeDtypeStruct((B, C, N), jnp.float32),
        mesh=plsc.VectorSubcoreMesh(core_axis_name="c", subcore_axis_name="s",
                                    num_cores=2, num_subcores=16),
        compiler_params=pltpu.CompilerParams(needs_layout_passes=False),
        scratch_types=[
            pltpu.VMEM((C, N), jnp.float32),
            pltpu.VMEM((GN * K,), jnp.int32),
            pltpu.VMEM((GN * K,), jnp.int32),
            pltpu.VMEM((C, GN), jnp.float32),
        ],
    )


_DOT = dict(preferred_element_type=jnp.float32, precision=lax.Precision.HIGHEST)


def _gelu(x):
    return 0.5 * x * (1.0 + lax.erf(x * 0.7071067811865476))


def _h1(x_ref, m_ref, wx_ref, wm_ref, b1_ref):
    return (jnp.dot(wx_ref[...], x_ref[0], **_DOT)
            + jnp.dot(wm_ref[...], m_ref[0], **_DOT) + b1_ref[...])


def _p1(x_ref, m_ref, wx_ref, wm_ref, b1_ref, s_ref):
    h = _h1(x_ref, m_ref, wx_ref, wm_ref, b1_ref)

    @pl.when(pl.program_id(0) == 0)
    def _():
        s_ref[...] = jnp.zeros_like(s_ref)

    s_ref[...] += jnp.concatenate(
        [jnp.sum(h, 1, keepdims=True), jnp.sum(h * h, 1, keepdims=True)], 1)


def _norm_gelu(h, s_ref, g_ref, be_ref):
    inv = 1.0 / NSAMP
    mean = s_ref[:, 0:1] * inv
    var = s_ref[:, 1:2] * inv - mean * mean
    scale = g_ref[...] * lax.rsqrt(var + 1e-5)
    return _gelu((h - mean) * scale + be_ref[...])


def _p2(x_ref, m_ref, wx_ref, wm_ref, b1_ref, s1_ref, g1_ref, be1_ref,
        w2_ref, b2_ref, h2_ref, s2_ref):
    h = _h1(x_ref, m_ref, wx_ref, wm_ref, b1_ref)
    g = _norm_gelu(h, s1_ref, g1_ref, be1_ref)
    h2 = jnp.dot(w2_ref[...], g, **_DOT) + b2_ref[...]
    h2_ref[0] = h2

    @pl.when(pl.program_id(0) == 0)
    def _():
        s2_ref[...] = jnp.zeros_like(s2_ref)

    s2_ref[...] += jnp.concatenate(
        [jnp.sum(h2, 1, keepdims=True), jnp.sum(h2 * h2, 1, keepdims=True)], 1)


def _p3(h2_ref, s2_ref, g2_ref, be2_ref, o_ref):
    o_ref[0] = _norm_gelu(h2_ref[0], s2_ref, g2_ref, be2_ref)


def _full(shape):
    return pl.BlockSpec(shape, lambda b: tuple(0 for _ in shape))


def kernel(x, edge_idx, conv1_w, conv1_b, bn1_g, bn1_b,
           conv2_w, conv2_b, bn2_g, bn2_b):
    xc = x.reshape(B, C, N)
    e0 = edge_idx[0].reshape(B, N * K)
    e1 = edge_idx[1].reshape(B, N * K)

    msg = _sc_msg()(xc, e0, e1)

    w1x = conv1_w[:, 0::2]
    w1m = conv1_w[:, 1::2]
    b1 = conv1_b.reshape(C2, 1)
    g1 = bn1_g.reshape(C2, 1)
    be1 = bn1_b.reshape(C2, 1)
    b2 = conv2_b.reshape(COUT, 1)
    g2 = bn2_g.reshape(COUT, 1)
    be2 = bn2_b.reshape(COUT, 1)

    xspec = pl.BlockSpec((1, C, N), lambda b: (b, 0, 0))
    sspec1 = _full((C2, 2))
    sspec2 = _full((COUT, 2))

    s1 = pl.pallas_call(
        _p1,
        grid=(B,),
        in_specs=[xspec, xspec, _full((C2, C)), _full((C2, C)), _full((C2, 1))],
        out_specs=sspec1,
        out_shape=jax.ShapeDtypeStruct((C2, 2), jnp.float32),
    )(xc, msg, w1x, w1m, b1)

    h2, s2 = pl.pallas_call(
        _p2,
        grid=(B,),
        in_specs=[xspec, xspec, _full((C2, C)), _full((C2, C)), _full((C2, 1)),
                  sspec1, _full((C2, 1)), _full((C2, 1)),
                  _full((COUT, C2)), _full((COUT, 1))],
        out_specs=[pl.BlockSpec((1, COUT, N), lambda b: (b, 0, 0)), sspec2],
        out_shape=[jax.ShapeDtypeStruct((B, COUT, N), jnp.float32),
                   jax.ShapeDtypeStruct((COUT, 2), jnp.float32)],
    )(xc, msg, w1x, w1m, b1, s1, g1, be1, conv2_w, b2)

    out = pl.pallas_call(
        _p3,
        grid=(B,),
        in_specs=[pl.BlockSpec((1, COUT, N), lambda b: (b, 0, 0)),
                  sspec2, _full((COUT, 1)), _full((COUT, 1))],
        out_specs=pl.BlockSpec((1, COUT, N), lambda b: (b, 0, 0)),
        out_shape=jax.ShapeDtypeStruct((B, COUT, N), jnp.float32),
    )(h2, s2, g2, be2)

    return out.reshape(B, COUT, 32, 32)
